# baseline (speedup 1.0000x reference)
import jax
import jax.numpy as jnp
from jax import lax
from jax.experimental import pallas as pl
from jax.experimental.pallas import tpu as pltpu
from jax.experimental.pallas import tpu_sc as plsc

B = 16384
T = 512

_LOG2C = (
    5.6422440275483154e-08,
    1.442685851294528,
    -0.7210957682030537,
    0.4781764415123899,
    -0.34542933660333985,
    0.2380419836756127,
    -0.13314692748387624,
    0.04943336843736993,
    -0.008665699320087797,
)
_LN2 = 0.6931471805599453
_LOG2E = 1.4426950408889634

_EXP2C = (
    0.9999980397841518,
    -0.6930489339094811,
    0.23943060364123772,
    -0.05321311778871329,
    0.006835154726653606,
)

_INFO = plsc.get_sparse_core_info()
_NC = _INFO.num_cores
_NS = _INFO.num_subcores
_NW = _NC * _NS
_RW = B // _NW
_CH = 64
_NCHUNK = _RW // _CH


def _treemul(vals):
    while len(vals) > 1:
        vals = [a * b for a, b in zip(vals[::2], vals[1::2])]
    return vals[0]


def _treeadd(vals):
    while len(vals) > 1:
        vals = [a + b for a, b in zip(vals[::2], vals[1::2])]
    return vals[0]


def _sc_kernel(phi_hbm, idx_hbm, ev_hbm, out_hbm,
               buf0, buf1, idx_v, ev_v, stage_v, sem0, sem1):
    wid = lax.axis_index("s") * _NC + lax.axis_index("c")
    base = wid * _RW

    pltpu.sync_copy(idx_hbm.at[pl.ds(base, _RW)], idx_v)
    pltpu.sync_copy(ev_hbm.at[pl.ds(base, _RW)], ev_v)

    bufs = (buf0, buf1)
    sems = (sem0, sem1)

    def start(c):
        return pltpu.async_copy(
            phi_hbm.at[pl.ds((base + c * _CH) * T, _CH * T)],
            bufs[c % 2], sems[c % 2])

    lane = lax.iota(jnp.int32, 16)
    copies = [None] * _NCHUNK
    copies[0] = start(0)
    acc_w = jnp.zeros((16,), jnp.float32)

    for c in range(_NCHUNK):
        if c + 1 < _NCHUNK:
            copies[c + 1] = start(c + 1)
        copies[c].wait()
        buf = bufs[c % 2]

        def group(g, acc_w):
            rowoff = (lane + g * 16) * T
            idx_vec = idx_v[pl.ds(c * _CH + g * 16, 16)]
            ev_vec = ev_v[pl.ds(c * _CH + g * 16, 16)]
            nsteps = jnp.max(idx_vec) // 16 + 1

            def step(s, carry):
                S, P, E = carry
                s16 = s * 16
                rem = idx_vec - s16
                base_idx = rowoff + s16
                fs = []
                ss = []
                for k in range(16):
                    x = plsc.load_gather(buf, [base_idx + k])
                    z = jnp.minimum(jnp.abs(x * _LOG2E), 30.0)
                    n = z.astype(jnp.int32)
                    r = z - n.astype(jnp.float32)
                    q = jnp.float32(_EXP2C[-1])
                    for coef in _EXP2C[-2::-1]:
                        q = coef + r * q
                    u = q * lax.bitcast_convert_type(
                        (127 - n) << 23, jnp.float32)
                    m = rem >= k
                    fs.append(jnp.where(m, 1.0 + u, 1.0))
                    ss.append(jnp.where(m, jnp.maximum(x, 0.0), 0.0))
                P = P * _treemul(fs)
                S = S + _treeadd(ss)
                bits = lax.bitcast_convert_type(P, jnp.int32)
                E = E + lax.shift_right_logical(bits, 23)
                P = lax.bitcast_convert_type(
                    (bits & 0x007FFFFF) | 0x3F800000, jnp.float32)
                return S, P, E

            S, P, E = lax.fori_loop(
                0, nsteps, step,
                (jnp.zeros((16,), jnp.float32),
                 jnp.ones((16,), jnp.float32),
                 jnp.zeros((16,), jnp.int32)))

            t = P - 1.0
            pl2 = jnp.float32(_LOG2C[-1])
            for coef in _LOG2C[-2::-1]:
                pl2 = coef + t * pl2
            ef = (E - 127 * nsteps).astype(jnp.float32)
            gathered = plsc.load_gather(buf, [rowoff + idx_vec])
            return acc_w + S + _LN2 * (ef + pl2) - ev_vec * gathered

        acc_w = lax.fori_loop(0, _CH // 16, group, acc_w)

    stage_v[...] = acc_w
    pltpu.sync_copy(stage_v, out_hbm.at[wid])


@jax.jit
def _run(phi, idx, ev):
    mesh = plsc.VectorSubcoreMesh(core_axis_name="c", subcore_axis_name="s")
    partials = pl.kernel(
        _sc_kernel,
        mesh=mesh,
        out_type=jax.ShapeDtypeStruct((_NW, 16), jnp.float32),
        scratch_types=[
            pltpu.VMEM((_CH * T,), jnp.float32),
            pltpu.VMEM((_CH * T,), jnp.float32),
            pltpu.VMEM((_RW,), jnp.int32),
            pltpu.VMEM((_RW,), jnp.float32),
            pltpu.VMEM((16,), jnp.float32),
            pltpu.SemaphoreType.DMA,
            pltpu.SemaphoreType.DMA,
        ],
        compiler_params=pltpu.CompilerParams(
            use_tc_tiling_on_sc=False, needs_layout_passes=False),
    )(phi.reshape(-1), idx, ev)
    return jnp.sum(partials) / B


def kernel(phi, idx_durations, events):
    return _run(phi, idx_durations.reshape(-1), events.reshape(-1))


# speedup vs baseline: 2.1223x; 2.1223x over previous
import jax
import jax.numpy as jnp
from jax import lax
from jax.experimental import pallas as pl
from jax.experimental.pallas import tpu as pltpu
from jax.experimental.pallas import tpu_sc as plsc

B = 16384
T = 512

_LOG2C = (
    5.6422440275483154e-08,
    1.442685851294528,
    -0.7210957682030537,
    0.4781764415123899,
    -0.34542933660333985,
    0.2380419836756127,
    -0.13314692748387624,
    0.04943336843736993,
    -0.008665699320087797,
)
_LN2 = 0.6931471805599453
_LOG2E = 1.4426950408889634

_EXP2C = (
    0.9999980397841518,
    -0.6930489339094811,
    0.23943060364123772,
    -0.05321311778871329,
    0.006835154726653606,
)

_INFO = plsc.get_sparse_core_info()
_NC = _INFO.num_cores
_NS = _INFO.num_subcores
_NW = _NC * _NS
_RW = B // _NW
_CH = 64
_NCHUNK = _RW // _CH


def _treemul(vals):
    while len(vals) > 1:
        vals = [a * b for a, b in zip(vals[::2], vals[1::2])]
    return vals[0]


def _treeadd(vals):
    while len(vals) > 1:
        vals = [a + b for a, b in zip(vals[::2], vals[1::2])]
    return vals[0]


def _sc_kernel(phi_hbm, idx_hbm, ev_hbm, out_hbm,
               buf0, buf1, idx_v, ev_v, stage_v, sem0, sem1):
    wid = lax.axis_index("s") * _NC + lax.axis_index("c")
    base = wid * _RW

    pltpu.sync_copy(idx_hbm.at[pl.ds(base, _RW)], idx_v)
    pltpu.sync_copy(ev_hbm.at[pl.ds(base, _RW)], ev_v)

    bufs = (buf0, buf1)
    sems = (sem0, sem1)

    def start(c):
        return pltpu.async_copy(
            phi_hbm.at[pl.ds((base + c * _CH) * T, _CH * T)],
            bufs[c % 2], sems[c % 2])

    lane = lax.iota(jnp.int32, 16)
    copies = [None] * _NCHUNK
    copies[0] = start(0)
    acc_w = jnp.zeros((16,), jnp.float32)

    for c in range(_NCHUNK):
        if c + 1 < _NCHUNK:
            copies[c + 1] = start(c + 1)
        copies[c].wait()
        buf = bufs[c % 2]

        def group(g, acc_w):
            rowoff = (lane + g * 16) * T
            idx_vec = idx_v[pl.ds(c * _CH + g * 16, 16)]
            ev_vec = ev_v[pl.ds(c * _CH + g * 16, 16)]
            nsteps = jnp.max(idx_vec) // 16 + 1
            colk = [(lane + k) & 15 for k in range(16)]

            def step(s, carry):
                S, P, E = carry
                s16 = s * 16
                rem = idx_vec - s16
                base_idx = rowoff + s16
                fs = []
                ss = []
                for k in range(16):
                    x = plsc.load_gather(buf, [base_idx + colk[k]])
                    u = jnp.exp(-jnp.abs(x))
                    m = rem >= colk[k]
                    fs.append(jnp.where(m, 1.0 + u, 1.0))
                    ss.append(jnp.where(m, jnp.maximum(x, 0.0), 0.0))
                P = P * _treemul(fs)
                S = S + _treeadd(ss)
                bits = lax.bitcast_convert_type(P, jnp.int32)
                E = E + lax.shift_right_logical(bits, 23)
                P = lax.bitcast_convert_type(
                    (bits & 0x007FFFFF) | 0x3F800000, jnp.float32)
                return S, P, E

            S, P, E = lax.fori_loop(
                0, nsteps, step,
                (jnp.zeros((16,), jnp.float32),
                 jnp.ones((16,), jnp.float32),
                 jnp.zeros((16,), jnp.int32)))

            t = P - 1.0
            pl2 = jnp.float32(_LOG2C[-1])
            for coef in _LOG2C[-2::-1]:
                pl2 = coef + t * pl2
            ef = (E - 127 * nsteps).astype(jnp.float32)
            gathered = plsc.load_gather(buf, [rowoff + idx_vec])
            return acc_w + S + _LN2 * (ef + pl2) - ev_vec * gathered

        acc_w = lax.fori_loop(0, _CH // 16, group, acc_w)

    stage_v[...] = acc_w
    pltpu.sync_copy(stage_v, out_hbm.at[wid])


@jax.jit
def _run(phi, idx, ev):
    mesh = plsc.VectorSubcoreMesh(core_axis_name="c", subcore_axis_name="s")
    partials = pl.kernel(
        _sc_kernel,
        mesh=mesh,
        out_type=jax.ShapeDtypeStruct((_NW, 16), jnp.float32),
        scratch_types=[
            pltpu.VMEM((_CH * T,), jnp.float32),
            pltpu.VMEM((_CH * T,), jnp.float32),
            pltpu.VMEM((_RW,), jnp.int32),
            pltpu.VMEM((_RW,), jnp.float32),
            pltpu.VMEM((16,), jnp.float32),
            pltpu.SemaphoreType.DMA,
            pltpu.SemaphoreType.DMA,
        ],
        compiler_params=pltpu.CompilerParams(
            use_tc_tiling_on_sc=False, needs_layout_passes=False),
    )(phi.reshape(-1), idx, ev)
    return jnp.sum(partials) / B


def kernel(phi, idx_durations, events):
    return _run(phi, idx_durations.reshape(-1), events.reshape(-1))
